# baseline (speedup 1.0000x reference)
import functools

import jax
import jax.numpy as jnp
from jax import lax
from jax.experimental import pallas as pl
from jax.experimental.pallas import tpu as pltpu
from jax.experimental.pallas import tpu_sc as plsc

_FIELD = 26
_FEAT = 10000
_K = 8
_D = _FIELD * _K
_BATCH = 4096

_NC = 2
_NS = 16
_NW = _NC * _NS
_RPW = _BATCH // _NW
_CB = 8
_NCHUNK = _RPW // _CB
_IDXN = _CB * _FIELD
_NF1 = 13
_SUBN = 112
_NV = _D // 16


def _dg(x, idx):
    return x.at[idx].get(mode="promise_in_bounds")


def _common_scratch():
    return [
        pltpu.VMEM((_IDXN,), jnp.int32),
        pltpu.VMEM((_IDXN,), jnp.int32),
        pltpu.VMEM((_IDXN,), jnp.int32),
        pltpu.VMEM((_SUBN,), jnp.int32),
        pltpu.VMEM((_SUBN,), jnp.int32),
        pltpu.VMEM((_SUBN,), jnp.int32),
        pltpu.VMEM((_SUBN,), jnp.int32),
        pltpu.VMEM((_SUBN,), jnp.int32),
        pltpu.VMEM((_SUBN,), jnp.int32),
        pltpu.VMEM((_SUBN, 128), jnp.float32),
        pltpu.VMEM((_SUBN, 128), jnp.float32),
        pltpu.VMEM((_SUBN, 128), jnp.float32),
        pltpu.VMEM((_SUBN, 128), jnp.float32),
        pltpu.VMEM((_CB * _D,), jnp.float32),
        pltpu.VMEM((_CB * _D,), jnp.float32),
        pltpu.SemaphoreType.DMA,
        pltpu.SemaphoreType.DMA,
    ]


def _worker_base():
    cid = lax.axis_index("c")
    sid = lax.axis_index("s")
    return (sid * _NC + cid) * _RPW


def _fill_patterns(off_v, pat_v, offp_v, fofs, sub_base=0, pad_f=0):
    lane = lax.iota(jnp.int32, 16)
    for j in range(_IDXN // 16):
        i16 = lane + (16 * j)
        off_v[pl.ds(16 * j, 16)] = (i16 % _FIELD) * _FEAT
    for t in range(_SUBN // 16):
        e = lane + (16 * t)
        r = lax.shift_right_logical(e * 5042, 16)
        pos = jnp.minimum(e + 13 * r + fofs, _IDXN - 1)
        pat_v[pl.ds(16 * t, 16)] = pos
        f_hat = jnp.where(e < _CB * _NF1, fofs + e - 13 * r, pad_f)
        offp_v[pl.ds(16 * t, 16)] = f_hat * _FEAT - sub_base


def _stage(c, base, bufs, in_hbm, va_hbm, vb_hbm, off_v, pat_v, offp_v,
           w_hbm=None, pin_hbm=None):
    idx_v, patc_v, sub_v, vra_v, vrb_v, wv_v, pacc_v, sem = bufs[c % 2]
    row0 = base + c * _CB
    for t in range(_SUBN // 16):
        sl = pl.ds(16 * t, 16)
        patc_v[sl] = pat_v[sl] + (row0 * _FIELD)
    cp_i = pltpu.async_copy(in_hbm.at[patc_v], sub_v, sem)
    cps = []
    if w_hbm is not None:
        pltpu.sync_copy(in_hbm.at[pl.ds(row0 * _FIELD, _IDXN)], idx_v)
        for j in range(_IDXN // 16):
            sl = pl.ds(16 * j, 16)
            idx_v[sl] = idx_v[sl] + off_v[sl]
        cps.append(pltpu.async_copy(w_hbm.at[idx_v], wv_v, sem))
    if pin_hbm is not None:
        cps.append(pltpu.async_copy(
            pin_hbm.at[pl.ds(row0 * _D, _CB * _D)], pacc_v, sem))
    cp_i.wait()
    for t in range(_SUBN // 16):
        sl = pl.ds(16 * t, 16)
        sub_v[sl] = sub_v[sl] + offp_v[sl]
    return [pltpu.async_copy(va_hbm.at[sub_v], vra_v, sem),
            pltpu.async_copy(vb_hbm.at[sub_v], vrb_v, sem)] + cps


def _accum(rb, vra_v, vrb_v, init):
    def f_body(f, accs):
        r = rb * _NF1 + f
        return tuple(
            (accs[j] + vra_v[r, pl.ds(16 * j, 16)]) if j < 8
            else (accs[j] + vrb_v[r, pl.ds(16 * (j - 8), 16)])
            for j in range(_NV))
    return lax.fori_loop(0, _NF1, f_body, init)


def _build_part1():
    mesh = plsc.VectorSubcoreMesh(core_axis_name="c", subcore_axis_name="s")

    @functools.partial(
        pl.kernel,
        mesh=mesh,
        compiler_params=pltpu.CompilerParams(use_tc_tiling_on_sc=False),
        out_type=jax.ShapeDtypeStruct((_BATCH * _D,), jnp.float32),
        scratch_types=_common_scratch(),
    )
    def ffm1(in_hbm, va_hbm, vb_hbm, pout_hbm,
             idx0_v, idx1_v, off_v, pat_v, offp_v, patc0_v, patc1_v,
             sub0_v, sub1_v,
             vra0_v, vra1_v, vrb0_v, vrb1_v, pacc0_v, pacc1_v, sem0, sem1):
        base = _worker_base()
        _fill_patterns(off_v, pat_v, offp_v, 0)
        bufs = [(idx0_v, patc0_v, sub0_v, vra0_v, vrb0_v, None, pacc0_v,
                 sem0),
                (idx1_v, patc1_v, sub1_v, vra1_v, vrb1_v, None, pacc1_v,
                 sem1)]

        cps = _stage(0, base, bufs, in_hbm, va_hbm, vb_hbm, off_v, pat_v,
                     offp_v)
        for c in range(_NCHUNK):
            vra_v, vrb_v, pacc_v = bufs[c % 2][3], bufs[c % 2][4], bufs[c % 2][6]
            nxt = (_stage(c + 1, base, bufs, in_hbm, va_hbm, vb_hbm,
                          off_v, pat_v, offp_v) if c + 1 < _NCHUNK else None)
            for cp in cps:
                cp.wait()
            cps = nxt

            for rb in range(_CB):
                init = tuple(jnp.zeros((16,), jnp.float32)
                             for _ in range(_NV))
                accs = _accum(rb, vra_v, vrb_v, init)
                for j in range(_NV):
                    pacc_v[pl.ds(rb * _D + 16 * j, 16)] = accs[j]

            row0 = base + c * _CB
            pltpu.sync_copy(pacc_v, pout_hbm.at[pl.ds(row0 * _D, _CB * _D)])

    return ffm1


def _build_part2():
    mesh = plsc.VectorSubcoreMesh(core_axis_name="c", subcore_axis_name="s")

    @functools.partial(
        pl.kernel,
        mesh=mesh,
        compiler_params=pltpu.CompilerParams(use_tc_tiling_on_sc=False),
        out_type=jax.ShapeDtypeStruct((_BATCH,), jnp.float32),
        scratch_types=_common_scratch() + [
            pltpu.VMEM((_IDXN,), jnp.float32),
            pltpu.VMEM((_IDXN,), jnp.float32),
            pltpu.VMEM((_RPW,), jnp.float32),
        ],
    )
    def ffm2(in_hbm, w_hbm, va_hbm, vb_hbm, pin_hbm, out_hbm,
             idx0_v, idx1_v, off_v, pat_v, offp_v, patc0_v, patc1_v,
             sub0_v, sub1_v,
             vra0_v, vra1_v, vrb0_v, vrb1_v, pacc0_v, pacc1_v, sem0, sem1,
             wv0_v, wv1_v, res_v):
        base = _worker_base()
        lane = lax.iota(jnp.int32, 16)
        _fill_patterns(off_v, pat_v, offp_v, _NF1,
                       sub_base=_TROFS * _TBLK, pad_f=_FIELD - 1)
        bufs = [(idx0_v, patc0_v, sub0_v, vra0_v, vrb0_v, wv0_v, pacc0_v,
                 sem0),
                (idx1_v, patc1_v, sub1_v, vra1_v, vrb1_v, wv1_v, pacc1_v,
                 sem1)]

        cps = _stage(0, base, bufs, in_hbm, va_hbm, vb_hbm, off_v, pat_v,
                     offp_v, w_hbm=w_hbm, pin_hbm=pin_hbm)
        for c in range(_NCHUNK):
            b = bufs[c % 2]
            vra_v, vrb_v, wv_v, pacc_v = b[3], b[4], b[5], b[6]
            nxt = (_stage(c + 1, base, bufs, in_hbm, va_hbm, vb_hbm,
                          off_v, pat_v, offp_v, w_hbm=w_hbm, pin_hbm=pin_hbm)
                   if c + 1 < _NCHUNK else None)
            for cp in cps:
                cp.wait()
            cps = nxt

            def row_body(rb, chunkres):
                init = tuple(pacc_v[pl.ds(rb * _D + 16 * j, 16)]
                             for j in range(_NV))
                accs = _accum(rb, vra_v, vrb_v, init)

                sq = accs[0] * accs[0]
                s16 = accs[0]
                for j in range(1, _NV):
                    sq = sq + accs[j] * accs[j]
                    s16 = s16 + accs[j]
                t = s16 + _dg(s16, lane ^ 8)

                e0 = rb * _FIELD
                a = pl.multiple_of((e0 // 8) * 8, 8)
                shift = e0 - a
                va = wv_v[pl.ds(a, 16)]
                vb = wv_v[pl.ds(a + 16, 16)]
                i1 = (shift + lane) & 15
                g1 = jnp.where(shift + lane < 16, _dg(va, i1), _dg(vb, i1))
                g2 = jnp.where(lane < 10, _dg(vb, i1),
                               jnp.zeros((16,), jnp.float32))

                combo = g1 + g2 + 0.25 * t * t - 0.5 * sq
                for sh in (8, 4, 2, 1):
                    combo = combo + _dg(combo, lane ^ sh)

                return jnp.where(lane == (rb + 8 * (c % 2)),
                                 combo, chunkres)

            if c % 2 == 0:
                chunkres0 = jnp.zeros((16,), jnp.float32)
            chunkres0 = lax.fori_loop(0, _CB, row_body, chunkres0)
            if c % 2 == 1:
                res_v[pl.ds((c // 2) * 16, 16)] = chunkres0

        pltpu.sync_copy(res_v, out_hbm.at[pl.ds(base, _RPW)])

    return ffm2


_TBLK = 8192
_TSPLIT = 16
_TROFS = 15


def _tr_body(vt_ref, a_ref, b_ref):
    xp = jnp.concatenate(
        [vt_ref[...], jnp.zeros((256 - _D, _TBLK), jnp.float32)], axis=0)
    xt = xp.T
    a_ref[...] = xt[:, :128]
    b_ref[...] = xt[:, 128:]


def _to_row_major(vt, nblk, ofs):
    rows = nblk * _TBLK
    return pl.pallas_call(
        _tr_body,
        grid=(nblk,),
        in_specs=[pl.BlockSpec((_D, _TBLK), lambda j: (0, j + ofs))],
        out_specs=[pl.BlockSpec((_TBLK, 128), lambda j: (j, 0)),
                   pl.BlockSpec((_TBLK, 128), lambda j: (j, 0))],
        out_shape=[jax.ShapeDtypeStruct((rows, 128), jnp.float32),
                   jax.ShapeDtypeStruct((rows, 128), jnp.float32)],
    )(vt)


def kernel(inputs, w0, w, v):
    rows = v.shape[0]
    vt = jnp.transpose(v, (1, 2, 0)).reshape(_D, rows)
    nblk = pl.cdiv(rows, _TBLK)
    a1, b1 = _to_row_major(vt, _TSPLIT, 0)
    part = _build_part1()(inputs.reshape(-1), a1, b1)
    a2, b2 = _to_row_major(vt, nblk - _TROFS, _TROFS)
    out = _build_part2()(inputs.reshape(-1), w.reshape(-1), a2, b2, part)
    return out.reshape(_BATCH, 1) + w0


# speedup vs baseline: 1.0193x; 1.0193x over previous
import functools

import jax
import jax.numpy as jnp
from jax import lax
from jax.experimental import pallas as pl
from jax.experimental.pallas import tpu as pltpu
from jax.experimental.pallas import tpu_sc as plsc

_FIELD = 26
_FEAT = 10000
_K = 8
_D = _FIELD * _K
_BATCH = 4096

_NC = 2
_NS = 16
_NW = _NC * _NS
_RPW = _BATCH // _NW
_CB = 8
_NCHUNK = _RPW // _CB
_IDXN = _CB * _FIELD
_NF1 = 13
_SUBN = 112
_NV = _D // 16


def _dg(x, idx):
    return x.at[idx].get(mode="promise_in_bounds")


def _common_scratch():
    return [
        pltpu.VMEM((_IDXN,), jnp.int32),
        pltpu.VMEM((_IDXN,), jnp.int32),
        pltpu.VMEM((_IDXN,), jnp.int32),
        pltpu.VMEM((_SUBN,), jnp.int32),
        pltpu.VMEM((_SUBN,), jnp.int32),
        pltpu.VMEM((_SUBN,), jnp.int32),
        pltpu.VMEM((_SUBN,), jnp.int32),
        pltpu.VMEM((_SUBN,), jnp.int32),
        pltpu.VMEM((_SUBN,), jnp.int32),
        pltpu.VMEM((_SUBN, 128), jnp.float32),
        pltpu.VMEM((_SUBN, 128), jnp.float32),
        pltpu.VMEM((_SUBN, 128), jnp.float32),
        pltpu.VMEM((_SUBN, 128), jnp.float32),
        pltpu.VMEM((_CB * _D,), jnp.float32),
        pltpu.VMEM((_CB * _D,), jnp.float32),
        pltpu.SemaphoreType.DMA,
        pltpu.SemaphoreType.DMA,
        pltpu.SemaphoreType.DMA,
        pltpu.SemaphoreType.DMA,
        pltpu.SemaphoreType.DMA,
        pltpu.SemaphoreType.DMA,
    ]


def _worker_base():
    cid = lax.axis_index("c")
    sid = lax.axis_index("s")
    return (sid * _NC + cid) * _RPW


def _fill_patterns(off_v, pat_v, offp_v, fofs, sub_base=0, pad_f=0):
    lane = lax.iota(jnp.int32, 16)
    for j in range(_IDXN // 16):
        i16 = lane + (16 * j)
        off_v[pl.ds(16 * j, 16)] = (i16 % _FIELD) * _FEAT
    for t in range(_SUBN // 16):
        e = lane + (16 * t)
        r = lax.shift_right_logical(e * 5042, 16)
        pos = jnp.minimum(e + 13 * r + fofs, _IDXN - 1)
        pat_v[pl.ds(16 * t, 16)] = pos
        f_hat = jnp.where(e < _CB * _NF1, fofs + e - 13 * r, pad_f)
        offp_v[pl.ds(16 * t, 16)] = f_hat * _FEAT - sub_base


def _stage(c, base, bufs, in_hbm, va_hbm, vb_hbm, off_v, pat_v, offp_v,
           w_hbm=None, pin_hbm=None):
    idx_v, patc_v, sub_v, vra_v, vrb_v, wv_v, pacc_v, sem = bufs[c % 2]
    row0 = base + c * _CB
    for t in range(_SUBN // 16):
        sl = pl.ds(16 * t, 16)
        patc_v[sl] = pat_v[sl] + (row0 * _FIELD)
    cp_i = pltpu.async_copy(in_hbm.at[patc_v], sub_v, sem)
    cps = []
    if w_hbm is not None:
        pltpu.sync_copy(in_hbm.at[pl.ds(row0 * _FIELD, _IDXN)], idx_v)
        for j in range(_IDXN // 16):
            sl = pl.ds(16 * j, 16)
            idx_v[sl] = idx_v[sl] + off_v[sl]
        cps.append(pltpu.async_copy(w_hbm.at[idx_v], wv_v, sem))
    if pin_hbm is not None:
        cps.append(pltpu.async_copy(
            pin_hbm.at[pl.ds(row0 * _D, _CB * _D)], pacc_v, sem))
    cp_i.wait()
    for t in range(_SUBN // 16):
        sl = pl.ds(16 * t, 16)
        sub_v[sl] = sub_v[sl] + offp_v[sl]
    return [pltpu.async_copy(va_hbm.at[sub_v], vra_v, sem),
            pltpu.async_copy(vb_hbm.at[sub_v], vrb_v, sem)] + cps


def _accum(rb, vra_v, vrb_v, init):
    def f_body(f, accs):
        r = rb * _NF1 + f
        return tuple(
            (accs[j] + vra_v[r, pl.ds(16 * j, 16)]) if j < 8
            else (accs[j] + vrb_v[r, pl.ds(16 * (j - 8), 16)])
            for j in range(_NV))
    return lax.fori_loop(0, _NF1, f_body, init)


def _build_part1():
    mesh = plsc.VectorSubcoreMesh(core_axis_name="c", subcore_axis_name="s")

    @functools.partial(
        pl.kernel,
        mesh=mesh,
        compiler_params=pltpu.CompilerParams(use_tc_tiling_on_sc=False),
        out_type=jax.ShapeDtypeStruct((_BATCH * _D,), jnp.float32),
        scratch_types=_common_scratch(),
    )
    def ffm1(in_hbm, va_hbm, vb_hbm, pout_hbm,
             idx0_v, idx1_v, off_v, pat_v, offp_v, patc0_v, patc1_v,
             sub0_v, sub1_v,
             vra0_v, vra1_v, vrb0_v, vrb1_v, pacc0_v, pacc1_v, sem0, sem1,
             semi0, semi1, semp0, semp1):
        base = _worker_base()
        _fill_patterns(off_v, pat_v, offp_v, 0)
        patc = [patc0_v, patc1_v]
        sub = [sub0_v, sub1_v]
        vra = [vra0_v, vra1_v]
        vrb = [vrb0_v, vrb1_v]
        pacc = [pacc0_v, pacc1_v]
        sem = [sem0, sem1]
        semi = [semi0, semi1]
        semp = [semp0, semp1]

        def fire_idx(c):
            p = c % 2
            row0 = base + c * _CB
            for t in range(_SUBN // 16):
                sl = pl.ds(16 * t, 16)
                patc[p][sl] = pat_v[sl] + (row0 * _FIELD)
            return pltpu.async_copy(in_hbm.at[patc[p]], sub[p], semi[p])

        def fire_ab(c, cpi):
            p = c % 2
            cpi.wait()
            for t in range(_SUBN // 16):
                sl = pl.ds(16 * t, 16)
                sub[p][sl] = sub[p][sl] + offp_v[sl]
            return (pltpu.async_copy(va_hbm.at[sub[p]], vra[p], sem[p]),
                    pltpu.async_copy(vb_hbm.at[sub[p]], vrb[p], sem[p]))

        cpi = {0: fire_idx(0), 1: fire_idx(1)}
        cpab = {0: fire_ab(0, cpi[0])}
        pw = {}
        for c in range(_NCHUNK):
            p = c % 2
            if c + 2 < _NCHUNK:
                cpi[c + 2] = fire_idx(c + 2)
            if c + 1 < _NCHUNK:
                cpab[c + 1] = fire_ab(c + 1, cpi[c + 1])
            for cp in cpab[c]:
                cp.wait()
            if c - 2 >= 0:
                pw[c - 2].wait()

            for rb in range(_CB):
                init = tuple(jnp.zeros((16,), jnp.float32)
                             for _ in range(_NV))
                accs = _accum(rb, vra[p], vrb[p], init)
                for j in range(_NV):
                    pacc[p][pl.ds(rb * _D + 16 * j, 16)] = accs[j]

            row0 = base + c * _CB
            pw[c] = pltpu.async_copy(
                pacc[p], pout_hbm.at[pl.ds(row0 * _D, _CB * _D)], semp[p])
        pw[_NCHUNK - 2].wait()
        pw[_NCHUNK - 1].wait()

    return ffm1


def _build_part2():
    mesh = plsc.VectorSubcoreMesh(core_axis_name="c", subcore_axis_name="s")

    @functools.partial(
        pl.kernel,
        mesh=mesh,
        compiler_params=pltpu.CompilerParams(use_tc_tiling_on_sc=False),
        out_type=jax.ShapeDtypeStruct((_BATCH,), jnp.float32),
        scratch_types=_common_scratch() + [
            pltpu.VMEM((_IDXN,), jnp.float32),
            pltpu.VMEM((_IDXN,), jnp.float32),
            pltpu.VMEM((_RPW,), jnp.float32),
        ],
    )
    def ffm2(in_hbm, w_hbm, va_hbm, vb_hbm, pin_hbm, out_hbm,
             idx0_v, idx1_v, off_v, pat_v, offp_v, patc0_v, patc1_v,
             sub0_v, sub1_v,
             vra0_v, vra1_v, vrb0_v, vrb1_v, pacc0_v, pacc1_v, sem0, sem1,
             semi0, semi1, semp0, semp1, wv0_v, wv1_v, res_v):
        del semi0, semi1, semp0, semp1
        base = _worker_base()
        lane = lax.iota(jnp.int32, 16)
        _fill_patterns(off_v, pat_v, offp_v, _NF1,
                       sub_base=_TROFS * _TBLK, pad_f=_FIELD - 1)
        bufs = [(idx0_v, patc0_v, sub0_v, vra0_v, vrb0_v, wv0_v, pacc0_v,
                 sem0),
                (idx1_v, patc1_v, sub1_v, vra1_v, vrb1_v, wv1_v, pacc1_v,
                 sem1)]

        cps = _stage(0, base, bufs, in_hbm, va_hbm, vb_hbm, off_v, pat_v,
                     offp_v, w_hbm=w_hbm, pin_hbm=pin_hbm)
        for c in range(_NCHUNK):
            b = bufs[c % 2]
            vra_v, vrb_v, wv_v, pacc_v = b[3], b[4], b[5], b[6]
            nxt = (_stage(c + 1, base, bufs, in_hbm, va_hbm, vb_hbm,
                          off_v, pat_v, offp_v, w_hbm=w_hbm, pin_hbm=pin_hbm)
                   if c + 1 < _NCHUNK else None)
            for cp in cps:
                cp.wait()
            cps = nxt

            def row_body(rb, chunkres):
                init = tuple(pacc_v[pl.ds(rb * _D + 16 * j, 16)]
                             for j in range(_NV))
                accs = _accum(rb, vra_v, vrb_v, init)

                sq = accs[0] * accs[0]
                s16 = accs[0]
                for j in range(1, _NV):
                    sq = sq + accs[j] * accs[j]
                    s16 = s16 + accs[j]
                t = s16 + _dg(s16, lane ^ 8)

                e0 = rb * _FIELD
                a = pl.multiple_of((e0 // 8) * 8, 8)
                shift = e0 - a
                va = wv_v[pl.ds(a, 16)]
                vb = wv_v[pl.ds(a + 16, 16)]
                i1 = (shift + lane) & 15
                g1 = jnp.where(shift + lane < 16, _dg(va, i1), _dg(vb, i1))
                g2 = jnp.where(lane < 10, _dg(vb, i1),
                               jnp.zeros((16,), jnp.float32))

                combo = g1 + g2 + 0.25 * t * t - 0.5 * sq
                for sh in (8, 4, 2, 1):
                    combo = combo + _dg(combo, lane ^ sh)

                return jnp.where(lane == (rb + 8 * (c % 2)),
                                 combo, chunkres)

            if c % 2 == 0:
                chunkres0 = jnp.zeros((16,), jnp.float32)
            chunkres0 = lax.fori_loop(0, _CB, row_body, chunkres0)
            if c % 2 == 1:
                res_v[pl.ds((c // 2) * 16, 16)] = chunkres0

        pltpu.sync_copy(res_v, out_hbm.at[pl.ds(base, _RPW)])

    return ffm2


_TBLK = 8192
_TSPLIT = 16
_TROFS = 15


def _tr_body(vt_ref, a_ref, b_ref):
    xp = jnp.concatenate(
        [vt_ref[...], jnp.zeros((256 - _D, _TBLK), jnp.float32)], axis=0)
    xt = xp.T
    a_ref[...] = xt[:, :128]
    b_ref[...] = xt[:, 128:]


def _to_row_major(vt, nblk, ofs):
    rows = nblk * _TBLK
    return pl.pallas_call(
        _tr_body,
        grid=(nblk,),
        in_specs=[pl.BlockSpec((_D, _TBLK), lambda j: (0, j + ofs))],
        out_specs=[pl.BlockSpec((_TBLK, 128), lambda j: (j, 0)),
                   pl.BlockSpec((_TBLK, 128), lambda j: (j, 0))],
        out_shape=[jax.ShapeDtypeStruct((rows, 128), jnp.float32),
                   jax.ShapeDtypeStruct((rows, 128), jnp.float32)],
    )(vt)


def kernel(inputs, w0, w, v):
    rows = v.shape[0]
    vt = jnp.transpose(v, (1, 2, 0)).reshape(_D, rows)
    nblk = pl.cdiv(rows, _TBLK)
    a1, b1 = _to_row_major(vt, _TSPLIT, 0)
    part = _build_part1()(inputs.reshape(-1), a1, b1)
    a2, b2 = _to_row_major(vt, nblk - _TROFS, _TROFS)
    out = _build_part2()(inputs.reshape(-1), w.reshape(-1), a2, b2, part)
    return out.reshape(_BATCH, 1) + w0


# speedup vs baseline: 1.1356x; 1.1142x over previous
import functools

import jax
import jax.numpy as jnp
from jax import lax
from jax.experimental import pallas as pl
from jax.experimental.pallas import tpu as pltpu
from jax.experimental.pallas import tpu_sc as plsc

_FIELD = 26
_FEAT = 10000
_K = 8
_D = _FIELD * _K
_BATCH = 4096

_NC = 2
_NS = 16
_NW = _NC * _NS
_RPW = _BATCH // _NW
_CB = 8
_NCHUNK = _RPW // _CB
_IDXN = _CB * _FIELD
_NV = _D // 16


def _dg(x, idx):
    return x.at[idx].get(mode="promise_in_bounds")


def _build():
    mesh = plsc.VectorSubcoreMesh(core_axis_name="c", subcore_axis_name="s")

    @functools.partial(
        pl.kernel,
        mesh=mesh,
        compiler_params=pltpu.CompilerParams(use_tc_tiling_on_sc=False),
        out_type=jax.ShapeDtypeStruct((_BATCH,), jnp.float32),
        scratch_types=[
            pltpu.VMEM((_IDXN,), jnp.int32),
            pltpu.VMEM((_IDXN,), jnp.int32),
            pltpu.VMEM((_IDXN,), jnp.int32),
            pltpu.VMEM((_IDXN, 128), jnp.float32),
            pltpu.VMEM((_IDXN, 128), jnp.float32),
            pltpu.VMEM((_IDXN, 128), jnp.float32),
            pltpu.VMEM((_IDXN, 128), jnp.float32),
            pltpu.VMEM((_IDXN,), jnp.float32),
            pltpu.VMEM((_IDXN,), jnp.float32),
            pltpu.VMEM((_RPW,), jnp.float32),
            pltpu.SemaphoreType.DMA,
            pltpu.SemaphoreType.DMA,
        ],
    )
    def ffm(in_hbm, w_hbm, va_hbm, vb_hbm, out_hbm,
            idx0_v, idx1_v, off_v, vra0_v, vra1_v, vrb0_v, vrb1_v,
            wv0_v, wv1_v, res_v, sem0, sem1):
        cid = lax.axis_index("c")
        sid = lax.axis_index("s")
        wid = sid * _NC + cid
        base = wid * _RPW
        lane = lax.iota(jnp.int32, 16)

        for j in range(_IDXN // 16):
            i16 = lane + (16 * j)
            off_v[pl.ds(16 * j, 16)] = (i16 % _FIELD) * _FEAT

        bufs = [(idx0_v, vra0_v, vrb0_v, wv0_v, sem0),
                (idx1_v, vra1_v, vrb1_v, wv1_v, sem1)]

        def stage(c):
            idx_v, vra_v, vrb_v, wv_v, sem = bufs[c % 2]
            row0 = base + c * _CB
            pltpu.sync_copy(in_hbm.at[pl.ds(row0 * _FIELD, _IDXN)], idx_v)
            for j in range(_IDXN // 16):
                sl = pl.ds(16 * j, 16)
                idx_v[sl] = idx_v[sl] + off_v[sl]
            return (pltpu.async_copy(va_hbm.at[idx_v], vra_v, sem),
                    pltpu.async_copy(vb_hbm.at[idx_v], vrb_v, sem),
                    pltpu.async_copy(w_hbm.at[idx_v], wv_v, sem))

        cps = stage(0)
        for c in range(_NCHUNK):
            _, vra_v, vrb_v, wv_v, _ = bufs[c % 2]
            nxt = stage(c + 1) if c + 1 < _NCHUNK else None
            for cp in cps:
                cp.wait()
            cps = nxt

            def row_body(rb, chunkres):
                def f_body(f, accs):
                    r = rb * _FIELD + f
                    return tuple(
                        (accs[j] + vra_v[r, pl.ds(16 * j, 16)]) if j < 8
                        else (accs[j] + vrb_v[r, pl.ds(16 * (j - 8), 16)])
                        for j in range(_NV))

                init = tuple(jnp.zeros((16,), jnp.float32)
                             for _ in range(_NV))
                accs = lax.fori_loop(0, _FIELD, f_body, init)

                sq = accs[0] * accs[0]
                s16 = accs[0]
                for j in range(1, _NV):
                    sq = sq + accs[j] * accs[j]
                    s16 = s16 + accs[j]
                t = s16 + _dg(s16, lane ^ 8)

                e0 = rb * _FIELD
                a = pl.multiple_of((e0 // 8) * 8, 8)
                shift = e0 - a
                va = wv_v[pl.ds(a, 16)]
                vb = wv_v[pl.ds(a + 16, 16)]
                i1 = (shift + lane) & 15
                g1 = jnp.where(shift + lane < 16, _dg(va, i1), _dg(vb, i1))
                g2 = jnp.where(lane < 10, _dg(vb, i1),
                               jnp.zeros((16,), jnp.float32))

                combo = g1 + g2 + 0.25 * t * t - 0.5 * sq
                for sh in (8, 4, 2, 1):
                    combo = combo + _dg(combo, lane ^ sh)

                return jnp.where(lane == (rb + 8 * (c % 2)),
                                 combo, chunkres)

            if c % 2 == 0:
                chunkres0 = jnp.zeros((16,), jnp.float32)
            chunkres0 = lax.fori_loop(0, _CB, row_body, chunkres0)
            if c % 2 == 1:
                res_v[pl.ds((c // 2) * 16, 16)] = chunkres0

        pltpu.sync_copy(res_v, out_hbm.at[pl.ds(base, _RPW)])

    return ffm


_TBLK = 8192


def _tr_body(vt_ref, a_ref, b_ref):
    xp = jnp.concatenate(
        [vt_ref[...], jnp.zeros((256 - _D, _TBLK), jnp.float32)], axis=0)
    xt = xp.T
    a_ref[...] = xt[:, :128]
    b_ref[...] = xt[:, 128:]


def _to_row_major(v):
    rows = v.shape[0]
    vt = jnp.transpose(v, (1, 2, 0)).reshape(_D, rows)
    return pl.pallas_call(
        _tr_body,
        grid=(pl.cdiv(rows, _TBLK),),
        in_specs=[pl.BlockSpec((_D, _TBLK), lambda j: (0, j))],
        out_specs=[pl.BlockSpec((_TBLK, 128), lambda j: (j, 0)),
                   pl.BlockSpec((_TBLK, 128), lambda j: (j, 0))],
        out_shape=[jax.ShapeDtypeStruct((rows, 128), jnp.float32),
                   jax.ShapeDtypeStruct((rows, 128), jnp.float32)],
    )(vt)


def kernel(inputs, w0, w, v):
    ffm = _build()
    va, vb = _to_row_major(v)
    out = ffm(inputs.reshape(-1), w.reshape(-1), va, vb)
    return out.reshape(_BATCH, 1) + w0


# speedup vs baseline: 1.1458x; 1.0090x over previous
import functools

import jax
import jax.numpy as jnp
from jax import lax
from jax.experimental import pallas as pl
from jax.experimental.pallas import tpu as pltpu
from jax.experimental.pallas import tpu_sc as plsc

_FIELD = 26
_FEAT = 10000
_K = 8
_D = _FIELD * _K
_BATCH = 4096

_NC = 2
_NS = 16
_NW = _NC * _NS
_RPW = _BATCH // _NW
_CB = 8
_NCHUNK = _RPW // _CB
_IDXN = _CB * _FIELD
_NV = _D // 16


def _dg(x, idx):
    return x.at[idx].get(mode="promise_in_bounds")


def _build():
    mesh = plsc.VectorSubcoreMesh(core_axis_name="c", subcore_axis_name="s")

    @functools.partial(
        pl.kernel,
        mesh=mesh,
        compiler_params=pltpu.CompilerParams(use_tc_tiling_on_sc=False),
        out_type=jax.ShapeDtypeStruct((_BATCH,), jnp.float32),
        scratch_types=[
            pltpu.VMEM((_RPW * _FIELD,), jnp.int32),
            pltpu.VMEM((_IDXN,), jnp.int32),
            pltpu.VMEM((_IDXN, 128), jnp.float32),
            pltpu.VMEM((_IDXN, 128), jnp.float32),
            pltpu.VMEM((_IDXN, 128), jnp.float32),
            pltpu.VMEM((_IDXN, 128), jnp.float32),
            pltpu.VMEM((_IDXN,), jnp.float32),
            pltpu.VMEM((_IDXN,), jnp.float32),
            pltpu.VMEM((_RPW,), jnp.float32),
            pltpu.SemaphoreType.DMA,
            pltpu.SemaphoreType.DMA,
        ],
    )
    def ffm(in_hbm, w_hbm, va_hbm, vb_hbm, out_hbm,
            idxall_v, off_v, vra0_v, vra1_v, vrb0_v, vrb1_v,
            wv0_v, wv1_v, res_v, sem0, sem1):
        cid = lax.axis_index("c")
        sid = lax.axis_index("s")
        wid = sid * _NC + cid
        base = wid * _RPW
        lane = lax.iota(jnp.int32, 16)

        for j in range(_IDXN // 16):
            i16 = lane + (16 * j)
            off_v[pl.ds(16 * j, 16)] = (i16 % _FIELD) * _FEAT

        pltpu.sync_copy(in_hbm.at[pl.ds(base * _FIELD, _RPW * _FIELD)],
                        idxall_v)
        for j in range(_RPW * _FIELD // 16):
            sl = pl.ds(16 * j, 16)
            idxall_v[sl] = idxall_v[sl] + off_v[pl.ds((j % 13) * 16, 16)]

        bufs = [(vra0_v, vrb0_v, wv0_v, sem0),
                (vra1_v, vrb1_v, wv1_v, sem1)]

        def stage(c):
            vra_v, vrb_v, wv_v, sem = bufs[c % 2]
            idx_v = idxall_v.at[pl.ds(c * _IDXN, _IDXN)]
            return (pltpu.async_copy(va_hbm.at[idx_v], vra_v, sem),
                    pltpu.async_copy(vb_hbm.at[idx_v], vrb_v, sem),
                    pltpu.async_copy(w_hbm.at[idx_v], wv_v, sem))

        cps = stage(0)
        for c in range(_NCHUNK):
            vra_v, vrb_v, wv_v, _ = bufs[c % 2]
            nxt = stage(c + 1) if c + 1 < _NCHUNK else None
            for cp in cps:
                cp.wait()
            cps = nxt

            def row_body(rb, chunkres):
                def f_body(f, accs):
                    r = rb * _FIELD + f
                    return tuple(
                        (accs[j] + vra_v[r, pl.ds(16 * j, 16)]) if j < 8
                        else (accs[j] + vrb_v[r, pl.ds(16 * (j - 8), 16)])
                        for j in range(_NV))

                init = tuple(jnp.zeros((16,), jnp.float32)
                             for _ in range(_NV))
                accs = lax.fori_loop(0, _FIELD, f_body, init)

                sq = accs[0] * accs[0]
                s16 = accs[0]
                for j in range(1, _NV):
                    sq = sq + accs[j] * accs[j]
                    s16 = s16 + accs[j]
                t = s16 + _dg(s16, lane ^ 8)

                e0 = rb * _FIELD
                a = pl.multiple_of((e0 // 8) * 8, 8)
                shift = e0 - a
                va = wv_v[pl.ds(a, 16)]
                vb = wv_v[pl.ds(a + 16, 16)]
                i1 = (shift + lane) & 15
                g1 = jnp.where(shift + lane < 16, _dg(va, i1), _dg(vb, i1))
                g2 = jnp.where(lane < 10, _dg(vb, i1),
                               jnp.zeros((16,), jnp.float32))

                combo = g1 + g2 + 0.25 * t * t - 0.5 * sq
                for sh in (8, 4, 2, 1):
                    combo = combo + _dg(combo, lane ^ sh)

                return jnp.where(lane == (rb + 8 * (c % 2)),
                                 combo, chunkres)

            if c % 2 == 0:
                chunkres0 = jnp.zeros((16,), jnp.float32)
            chunkres0 = lax.fori_loop(0, _CB, row_body, chunkres0)
            if c % 2 == 1:
                res_v[pl.ds((c // 2) * 16, 16)] = chunkres0

        pltpu.sync_copy(res_v, out_hbm.at[pl.ds(base, _RPW)])

    return ffm


_TBLK = 8192


def _tr_body(vt_ref, a_ref, b_ref):
    xp = jnp.concatenate(
        [vt_ref[...], jnp.zeros((256 - _D, _TBLK), jnp.float32)], axis=0)
    xt = xp.T
    a_ref[...] = xt[:, :128]
    b_ref[...] = xt[:, 128:]


def _to_row_major(v):
    rows = v.shape[0]
    vt = jnp.transpose(v, (1, 2, 0)).reshape(_D, rows)
    return pl.pallas_call(
        _tr_body,
        grid=(pl.cdiv(rows, _TBLK),),
        in_specs=[pl.BlockSpec((_D, _TBLK), lambda j: (0, j))],
        out_specs=[pl.BlockSpec((_TBLK, 128), lambda j: (j, 0)),
                   pl.BlockSpec((_TBLK, 128), lambda j: (j, 0))],
        out_shape=[jax.ShapeDtypeStruct((rows, 128), jnp.float32),
                   jax.ShapeDtypeStruct((rows, 128), jnp.float32)],
    )(vt)


def kernel(inputs, w0, w, v):
    ffm = _build()
    va, vb = _to_row_major(v)
    out = ffm(inputs.reshape(-1), w.reshape(-1), va, vb)
    return out.reshape(_BATCH, 1) + w0
